# baseline (speedup 1.0000x reference)
import functools

import jax
import jax.numpy as jnp
import numpy as np
from jax import lax
from jax.experimental import pallas as pl
from jax.experimental.pallas import tpu as pltpu
from jax.experimental.pallas import tpu_sc as plsc

N = 10000
D = 128
E = 320000

NC = 2
NS = 16
NW = NC * NS
EW = E // NW
K = 80
CHUNKS = EW // K
RPT = 624
ZR = 48
TAIL = N - NS * RPT
PAIRS = CHUNKS // 2
UNROLL = 5
EPAD = 2 * K

_MAGIC = 0x5F3759DF

_PI = np.concatenate(
    [np.concatenate([np.arange(32 * q, 32 * q + 32, 2),
                     np.arange(32 * q + 1, 32 * q + 32, 2)])
     for q in range(D // 32)])

_GATHER_DNUMS = lax.GatherDimensionNumbers(
    offset_dims=(), collapsed_slice_dims=(0,), start_index_map=(0,))


def _shuffle(v, idx):
    return lax.gather(v, idx[:, None], dimension_numbers=_GATHER_DNUMS,
                      slice_sizes=(1,),
                      mode=lax.GatherScatterMode.PROMISE_IN_BOUNDS)


def _lane_total(v, shuffle_idx):
    for idx in shuffle_idx:
        v = v + _shuffle(v, idx)
    return v


_BLK = 1000


def _norm_rows(h):
    n = jnp.sqrt(jnp.sum(h * h, axis=1, keepdims=True))
    return h / (n + 1e-8)


def _dense_in_body(x_ref, w_ref, b_ref, o_ref):
    h = jnp.dot(x_ref[...], w_ref[...], preferred_element_type=jnp.float32)
    h = jnp.maximum(h + b_ref[...], 0.0)
    o_ref[...] = _norm_rows(h)


def _dense_mid_body(p0_ref, p1_ref, w_ref, b_ref, o_ref):
    h = p0_ref[...] + p1_ref[...]
    h = jnp.dot(h, w_ref[...], preferred_element_type=jnp.float32)
    h = jnp.maximum(h + b_ref[...], 0.0)
    o_ref[...] = _norm_rows(h)


def _head_body(p0_ref, p1_ref, w2_ref, b2_ref, wc1_ref, bc1_ref, wc2_ref,
               bc2_ref, o_ref):
    h = p0_ref[...] + p1_ref[...]
    h = jnp.dot(h, w2_ref[...], preferred_element_type=jnp.float32)
    h = jnp.maximum(h + b2_ref[...], 0.0)
    h = _norm_rows(h)
    c = jnp.dot(h, wc1_ref[...], preferred_element_type=jnp.float32)
    c = jnp.maximum(c + bc1_ref[...], 0.0)
    o = jnp.dot(c, wc2_ref[...], preferred_element_type=jnp.float32)
    o_ref[...] = jax.nn.sigmoid(o + bc2_ref[...])


def _row_spec(cols):
    return pl.BlockSpec((_BLK, cols), lambda i: (i, 0))


def _full_spec(r, c):
    return pl.BlockSpec((r, c), lambda i: (0, 0))


def _dense_in(x, wt, b):
    return pl.pallas_call(
        _dense_in_body,
        grid=(N // _BLK,),
        in_specs=[_row_spec(D), _full_spec(*wt.shape), _full_spec(*b.shape)],
        out_specs=_row_spec(wt.shape[1]),
        out_shape=jax.ShapeDtypeStruct((N, wt.shape[1]), jnp.float32),
    )(x, wt, b)


def _dense_mid(p0, p1, wt, b):
    return pl.pallas_call(
        _dense_mid_body,
        grid=(N // _BLK,),
        in_specs=[_row_spec(D), _row_spec(D), _full_spec(*wt.shape),
                  _full_spec(*b.shape)],
        out_specs=_row_spec(wt.shape[1]),
        out_shape=jax.ShapeDtypeStruct((N, wt.shape[1]), jnp.float32),
    )(p0, p1, wt, b)


def _head(p0, p1, w2t, b2, wc1t, bc1, wc2t, bc2):
    return pl.pallas_call(
        _head_body,
        grid=(N // _BLK,),
        in_specs=[_row_spec(D), _row_spec(D),
                  _full_spec(*w2t.shape), _full_spec(*b2.shape),
                  _full_spec(*wc1t.shape), _full_spec(*bc1.shape),
                  _full_spec(*wc2t.shape), _full_spec(*bc2.shape)],
        out_specs=_row_spec(1),
        out_shape=jax.ShapeDtypeStruct((N, 1), jnp.float32),
    )(p0, p1, w2t, b2, wc1t, bc1, wc2t, bc2)


def _mp_body(h_hbm, row_hbm, col_hbm, out_hbm,
             row_a, col_a, src_a, tgt_a, row_b, col_b, src_b, tgt_b,
             scol_a, scol_b, msg_a, msg_b, zero_v, acc_sh,
             sem_ia, sem_ib, sem_ga, sem_gb, sem_sa, sem_sb, sem_z):
    c = lax.axis_index("c")
    s = lax.axis_index("s")

    def zero_row(i, carry):
        for j in range(D // 16):
            zero_v[i, pl.ds(j * 16, 16)] = jnp.zeros((16,), jnp.float32)
        return carry

    lax.fori_loop(0, ZR, zero_row, 0)
    for k in range(RPT // ZR):
        pltpu.async_copy(zero_v, acc_sh.at[pl.ds(s * RPT + k * ZR, ZR)],
                         sem_z)

    @pl.when(s == 0)
    def _zero_tail():
        pltpu.async_copy(zero_v.at[pl.ds(0, TAIL)],
                         acc_sh.at[pl.ds(NS * RPT, TAIL)], sem_z)

    ebase = (c * NS + s) * EW
    lane = lax.iota(jnp.int32, 16)
    shuffle_idx = [lax.bitwise_xor(lane, jnp.int32(1 << k)) for k in range(4)]

    bufs = ((row_a, col_a, src_a, tgt_a, sem_ia, sem_ga, scol_a, sem_sa,
             msg_a),
            (row_b, col_b, src_b, tgt_b, sem_ib, sem_gb, scol_b, sem_sb,
             msg_b))

    def idx_start(base, buf):
        rv, cv, sem_i = buf[0], buf[1], buf[4]
        pltpu.async_copy(row_hbm.at[pl.ds(base, K)], rv, sem_i)
        pltpu.async_copy(col_hbm.at[pl.ds(base, K)], cv, sem_i)

    def idx_wait(buf):
        rv, cv, sem_i = buf[0], buf[1], buf[4]
        pltpu.make_async_copy(row_hbm.at[pl.ds(0, K)], rv, sem_i).wait()
        pltpu.make_async_copy(col_hbm.at[pl.ds(0, K)], cv, sem_i).wait()

    def gather_start(buf):
        rv, cv, sv, tv, sem_g = buf[0], buf[1], buf[2], buf[3], buf[5]
        pltpu.async_copy(h_hbm.at[rv], sv, sem_g)
        pltpu.async_copy(h_hbm.at[cv], tv, sem_g)

    def gather_wait(buf):
        rv, cv, sv, tv, sem_g = buf[0], buf[1], buf[2], buf[3], buf[5]
        pltpu.make_async_copy(h_hbm.at[rv], sv, sem_g).wait()
        pltpu.make_async_copy(h_hbm.at[cv], tv, sem_g).wait()

    def col_snapshot(buf):
        cv, scv = buf[1], buf[6]
        for m in range(K // 16):
            scv[pl.ds(m * 16, 16)] = cv[pl.ds(m * 16, 16)]

    hi_mask = jnp.full((16,), -65536, jnp.int32)

    def _decode(word):
        lo = lax.bitcast_convert_type(lax.shift_left(word, 16), jnp.float32)
        hi = lax.bitcast_convert_type(lax.bitwise_and(word, hi_mask),
                                      jnp.float32)
        return lo, hi

    def compute(buf):
        sv, tv, mv = buf[2], buf[3], buf[8]

        def edge_group(g, ecarry):
            e0 = g * UNROLL
            svals = [[] for _ in range(UNROLL)]
            accs = []
            for u in range(UNROLL):
                e = e0 + u
                parts = []
                for j in range(D // 32):
                    swv = sv[e, pl.ds(j * 16, 16)]
                    twv = tv[e, pl.ds(j * 16, 16)]
                    s_lo, s_hi = _decode(swv)
                    t_lo, t_hi = _decode(twv)
                    d0 = s_lo - t_lo
                    d1 = s_hi - t_hi
                    parts.append(d0 * d0)
                    parts.append(d1 * d1)
                    svals[u].append(s_lo)
                    svals[u].append(s_hi)
                while len(parts) > 1:
                    parts = [a + b for a, b in zip(parts[::2], parts[1::2])]
                accs.append(parts[0])
            ws = []
            for u in range(UNROLL):
                xv = _lane_total(accs[u], shuffle_idx) + 1e-12
                ii = lax.bitcast_convert_type(xv, jnp.int32)
                ii = _MAGIC - lax.shift_right_logical(ii, 1)
                y = lax.bitcast_convert_type(ii, jnp.float32)
                for _ in range(2):
                    y = y * (1.5 - 0.5 * xv * y * y)
                ws.append(jnp.exp(-(xv * y)))
            for u in range(UNROLL):
                e = e0 + u
                for j in range(D // 16):
                    mv[e, pl.ds(j * 16, 16)] = svals[u][j] * ws[u]
            return ecarry

        lax.fori_loop(0, K // UNROLL, edge_group, 0)

    def scatter_start(buf):
        mv, scv, sem_s = buf[8], buf[6], buf[7]
        pltpu.async_copy(mv, acc_sh.at[scv], sem_s, add=True)

    def scatter_wait(buf):
        mv, scv, sem_s = buf[8], buf[6], buf[7]
        pltpu.make_async_copy(mv, acc_sh.at[scv], sem_s).wait()

    def phase(this, other, t_pre, first=False):
        gather_wait(this)
        idx_wait(other)
        if not first:
            scatter_wait(other)
        gather_start(other)
        col_snapshot(this)
        compute(this)
        scatter_start(this)
        idx_start(ebase + t_pre * K, this)

    idx_start(ebase, bufs[0])
    idx_wait(bufs[0])
    gather_start(bufs[0])
    idx_start(ebase + K, bufs[1])

    for k in range(RPT // ZR):
        pltpu.make_async_copy(zero_v, acc_sh.at[pl.ds(s * RPT + k * ZR, ZR)],
                              sem_z).wait()

    @pl.when(s == 0)
    def _zero_tail_wait():
        pltpu.make_async_copy(zero_v.at[pl.ds(0, TAIL)],
                              acc_sh.at[pl.ds(NS * RPT, TAIL)], sem_z).wait()

    plsc.subcore_barrier()
    phase(bufs[0], bufs[1], 2, first=True)
    phase(bufs[1], bufs[0], 3)

    def pair(i, carry):
        t0 = 2 * i
        phase(bufs[0], bufs[1], t0 + 2)
        phase(bufs[1], bufs[0], t0 + 3)
        return carry

    lax.fori_loop(1, PAIRS, pair, 0)

    gather_wait(bufs[0])
    col_snapshot(bufs[0])
    compute(bufs[0])
    scatter_wait(bufs[1])
    scatter_start(bufs[0])
    idx_wait(bufs[1])
    scatter_wait(bufs[0])

    plsc.subcore_barrier()
    pltpu.sync_copy(acc_sh.at[pl.ds(s * RPT, RPT)],
                    out_hbm.at[c, pl.ds(s * RPT, RPT)])

    @pl.when(s == 0)
    def _write_tail():
        pltpu.sync_copy(acc_sh.at[pl.ds(NS * RPT, TAIL)],
                        out_hbm.at[c, pl.ds(NS * RPT, TAIL)])


@functools.partial(
    pl.kernel,
    out_type=jax.ShapeDtypeStruct((NC, N, D), jnp.float32),
    mesh=plsc.VectorSubcoreMesh(core_axis_name="c", subcore_axis_name="s"),
    compiler_params=pltpu.CompilerParams(use_tc_tiling_on_sc=False),
    scratch_types=[
        pltpu.VMEM((K,), jnp.int32),
        pltpu.VMEM((K,), jnp.int32),
        pltpu.VMEM((K, D // 2), jnp.int32),
        pltpu.VMEM((K, D // 2), jnp.int32),
        pltpu.VMEM((K,), jnp.int32),
        pltpu.VMEM((K,), jnp.int32),
        pltpu.VMEM((K, D // 2), jnp.int32),
        pltpu.VMEM((K, D // 2), jnp.int32),
        pltpu.VMEM((K,), jnp.int32),
        pltpu.VMEM((K,), jnp.int32),
        pltpu.VMEM((K, D), jnp.float32),
        pltpu.VMEM((K, D), jnp.float32),
        pltpu.VMEM((ZR, D), jnp.float32),
        pltpu.VMEM_SHARED((N, D), jnp.float32),
        pltpu.SemaphoreType.DMA,
        pltpu.SemaphoreType.DMA,
        pltpu.SemaphoreType.DMA,
        pltpu.SemaphoreType.DMA,
        pltpu.SemaphoreType.DMA,
        pltpu.SemaphoreType.DMA,
        pltpu.SemaphoreType.DMA,
    ],
)
def _message_pass(h_hbm, row_hbm, col_hbm, out_hbm,
                  row_a, col_a, src_a, tgt_a, row_b, col_b, src_b, tgt_b,
                  scol_a, scol_b, msg_a, msg_b, zero_v, acc_sh,
                  sem_ia, sem_ib, sem_ga, sem_gb, sem_sa, sem_sb, sem_z):
    _mp_body(h_hbm, row_hbm, col_hbm, out_hbm,
             row_a, col_a, src_a, tgt_a, row_b, col_b, src_b, tgt_b,
             scol_a, scol_b, msg_a, msg_b, zero_v, acc_sh,
             sem_ia, sem_ib, sem_ga, sem_gb, sem_sa, sem_sb, sem_z)


def _pack_bf16(h):
    hb = h.astype(jnp.bfloat16).reshape(N, D // 2, 2)
    return jax.lax.bitcast_convert_type(hb, jnp.int32)


def kernel(x, edge_index, W_in, b_in, W1, b1, W2, b2, Wc1, bc1, Wc2, bc2):
    ei = edge_index.astype(jnp.int32)
    pad = jnp.zeros((EPAD,), jnp.int32)
    row = jnp.concatenate([ei[0], pad])
    col = jnp.concatenate([ei[1], pad])

    h0 = _dense_in(x, W_in.T, b_in.reshape(1, -1))
    p = _message_pass(_pack_bf16(h0), row, col)
    h1 = _dense_mid(p[0], p[1], W1.T[_PI], b1.reshape(1, -1))
    p = _message_pass(_pack_bf16(h1), row, col)
    return _head(p[0], p[1], W2.T[_PI], b2.reshape(1, -1), Wc1.T,
                 bc1.reshape(1, -1), Wc2.T, bc2.reshape(1, -1))


# speedup vs baseline: 1.1149x; 1.1149x over previous
import functools

import jax
import jax.numpy as jnp
from jax import lax
from jax.experimental import pallas as pl
from jax.experimental.pallas import tpu as pltpu
from jax.experimental.pallas import tpu_sc as plsc

N = 10000
D = 128
E = 320000

NC = 2
NS = 16
NW = NC * NS
EW = E // NW
K = 80
CHUNKS = EW // K
RPT = 624
ZR = 16
TAIL = N - NS * RPT
PAIRS = CHUNKS // 2
UNROLL = 8
EPAD = 2 * K

_MAGIC = 0x5F3759DF

_GATHER_DNUMS = lax.GatherDimensionNumbers(
    offset_dims=(), collapsed_slice_dims=(0,), start_index_map=(0,))


def _shuffle(v, idx):
    return lax.gather(v, idx[:, None], dimension_numbers=_GATHER_DNUMS,
                      slice_sizes=(1,),
                      mode=lax.GatherScatterMode.PROMISE_IN_BOUNDS)


def _lane_total(v, shuffle_idx):
    for idx in shuffle_idx:
        v = v + _shuffle(v, idx)
    return v


_BLK = 1000


def _norm_rows(h):
    n = jnp.sqrt(jnp.sum(h * h, axis=1, keepdims=True))
    return h / (n + 1e-8)


def _dense_in_body(x_ref, w_ref, b_ref, o_ref):
    h = jnp.dot(x_ref[...], w_ref[...], preferred_element_type=jnp.float32)
    h = jnp.maximum(h + b_ref[...], 0.0)
    o_ref[...] = _norm_rows(h)


def _dense_mid_body(p0_ref, p1_ref, w_ref, b_ref, o_ref):
    h = p0_ref[...] + p1_ref[...]
    h = jnp.dot(h, w_ref[...], preferred_element_type=jnp.float32)
    h = jnp.maximum(h + b_ref[...], 0.0)
    o_ref[...] = _norm_rows(h)


def _head_body(p0_ref, p1_ref, w2_ref, b2_ref, wc1_ref, bc1_ref, wc2_ref,
               bc2_ref, o_ref):
    h = p0_ref[...] + p1_ref[...]
    h = jnp.dot(h, w2_ref[...], preferred_element_type=jnp.float32)
    h = jnp.maximum(h + b2_ref[...], 0.0)
    h = _norm_rows(h)
    c = jnp.dot(h, wc1_ref[...], preferred_element_type=jnp.float32)
    c = jnp.maximum(c + bc1_ref[...], 0.0)
    o = jnp.dot(c, wc2_ref[...], preferred_element_type=jnp.float32)
    o_ref[...] = jax.nn.sigmoid(o + bc2_ref[...])


def _row_spec(cols):
    return pl.BlockSpec((_BLK, cols), lambda i: (i, 0))


def _full_spec(r, c):
    return pl.BlockSpec((r, c), lambda i: (0, 0))


def _dense_in(x, wt, b):
    return pl.pallas_call(
        _dense_in_body,
        grid=(N // _BLK,),
        in_specs=[_row_spec(D), _full_spec(*wt.shape), _full_spec(*b.shape)],
        out_specs=_row_spec(wt.shape[1]),
        out_shape=jax.ShapeDtypeStruct((N, wt.shape[1]), jnp.float32),
    )(x, wt, b)


def _dense_mid(p0, p1, wt, b):
    return pl.pallas_call(
        _dense_mid_body,
        grid=(N // _BLK,),
        in_specs=[_row_spec(D), _row_spec(D), _full_spec(*wt.shape),
                  _full_spec(*b.shape)],
        out_specs=_row_spec(wt.shape[1]),
        out_shape=jax.ShapeDtypeStruct((N, wt.shape[1]), jnp.float32),
    )(p0, p1, wt, b)


def _head(p0, p1, w2t, b2, wc1t, bc1, wc2t, bc2):
    return pl.pallas_call(
        _head_body,
        grid=(N // _BLK,),
        in_specs=[_row_spec(D), _row_spec(D),
                  _full_spec(*w2t.shape), _full_spec(*b2.shape),
                  _full_spec(*wc1t.shape), _full_spec(*bc1.shape),
                  _full_spec(*wc2t.shape), _full_spec(*bc2.shape)],
        out_specs=_row_spec(1),
        out_shape=jax.ShapeDtypeStruct((N, 1), jnp.float32),
    )(p0, p1, w2t, b2, wc1t, bc1, wc2t, bc2)


def _mp_body(h_hbm, row_hbm, col_hbm, out_hbm,
             row_a, col_a, src_a, tgt_a, row_b, col_b, src_b, tgt_b,
             scol_a, scol_b, zero_v, acc_sh,
             sem_ia, sem_ib, sem_ga, sem_gb, sem_sa, sem_sb, sem_z):
    c = lax.axis_index("c")
    s = lax.axis_index("s")

    def zero_row(i, carry):
        for j in range(D // 16):
            zero_v[i, pl.ds(j * 16, 16)] = jnp.zeros((16,), jnp.float32)
        return carry

    lax.fori_loop(0, ZR, zero_row, 0)
    for k in range(RPT // ZR):
        pltpu.async_copy(zero_v, acc_sh.at[pl.ds(s * RPT + k * ZR, ZR)],
                         sem_z)

    @pl.when(s == 0)
    def _zero_tail():
        pltpu.async_copy(zero_v.at[pl.ds(0, TAIL)],
                         acc_sh.at[pl.ds(NS * RPT, TAIL)], sem_z)

    ebase = (c * NS + s) * EW
    lane = lax.iota(jnp.int32, 16)
    shuffle_idx = [lax.bitwise_xor(lane, jnp.int32(1 << k)) for k in range(4)]

    bufs = ((row_a, col_a, src_a, tgt_a, sem_ia, sem_ga, scol_a, sem_sa),
            (row_b, col_b, src_b, tgt_b, sem_ib, sem_gb, scol_b, sem_sb))

    def idx_start(base, buf):
        rv, cv, sem_i = buf[0], buf[1], buf[4]
        pltpu.async_copy(row_hbm.at[pl.ds(base, K)], rv, sem_i)
        pltpu.async_copy(col_hbm.at[pl.ds(base, K)], cv, sem_i)

    def idx_wait(buf):
        rv, cv, sem_i = buf[0], buf[1], buf[4]
        pltpu.make_async_copy(row_hbm.at[pl.ds(0, K)], rv, sem_i).wait()
        pltpu.make_async_copy(col_hbm.at[pl.ds(0, K)], cv, sem_i).wait()

    def gather_start(buf):
        rv, cv, sv, tv, sem_g = buf[0], buf[1], buf[2], buf[3], buf[5]
        pltpu.async_copy(h_hbm.at[rv], sv, sem_g)
        pltpu.async_copy(h_hbm.at[cv], tv, sem_g)

    def gather_wait(buf):
        rv, cv, sv, tv, sem_g = buf[0], buf[1], buf[2], buf[3], buf[5]
        pltpu.make_async_copy(h_hbm.at[rv], sv, sem_g).wait()
        pltpu.make_async_copy(h_hbm.at[cv], tv, sem_g).wait()

    def col_snapshot(buf):
        cv, scv = buf[1], buf[6]
        for m in range(K // 16):
            scv[pl.ds(m * 16, 16)] = cv[pl.ds(m * 16, 16)]

    def compute(buf):
        sv, tv = buf[2], buf[3]

        def edge_group(g, ecarry):
            e0 = g * UNROLL
            svals = [[] for _ in range(UNROLL)]
            accs = []
            for u in range(UNROLL):
                e = e0 + u
                parts = []
                for j in range(D // 16):
                    sj = sv[e, pl.ds(j * 16, 16)]
                    tj = tv[e, pl.ds(j * 16, 16)]
                    d = sj - tj
                    parts.append(d * d)
                    svals[u].append(sj)
                while len(parts) > 1:
                    parts = [a + b for a, b in zip(parts[::2], parts[1::2])]
                accs.append(parts[0])
            ws = []
            for u in range(UNROLL):
                xv = _lane_total(accs[u], shuffle_idx) + 1e-12
                ii = lax.bitcast_convert_type(xv, jnp.int32)
                ii = _MAGIC - lax.shift_right_logical(ii, 1)
                y = lax.bitcast_convert_type(ii, jnp.float32)
                for _ in range(2):
                    y = y * (1.5 - 0.5 * xv * y * y)
                ws.append(jnp.exp(-(xv * y)))
            for u in range(UNROLL):
                e = e0 + u
                for j in range(D // 16):
                    sv[e, pl.ds(j * 16, 16)] = svals[u][j] * ws[u]
            return ecarry

        lax.fori_loop(0, K // UNROLL, edge_group, 0)

    def scatter_start(buf):
        sv, scv, sem_s = buf[2], buf[6], buf[7]
        pltpu.async_copy(sv, acc_sh.at[scv], sem_s, add=True)

    def scatter_wait(buf):
        sv, scv, sem_s = buf[2], buf[6], buf[7]
        pltpu.make_async_copy(sv, acc_sh.at[scv], sem_s).wait()

    def phase(this, other, t_pre, first=False):
        gather_wait(this)
        idx_wait(other)
        if not first:
            scatter_wait(other)
        gather_start(other)
        col_snapshot(this)
        compute(this)
        scatter_start(this)
        idx_start(ebase + t_pre * K, this)

    idx_start(ebase, bufs[0])
    idx_wait(bufs[0])
    gather_start(bufs[0])
    idx_start(ebase + K, bufs[1])

    for k in range(RPT // ZR):
        pltpu.make_async_copy(zero_v, acc_sh.at[pl.ds(s * RPT + k * ZR, ZR)],
                              sem_z).wait()

    @pl.when(s == 0)
    def _zero_tail_wait():
        pltpu.make_async_copy(zero_v.at[pl.ds(0, TAIL)],
                              acc_sh.at[pl.ds(NS * RPT, TAIL)], sem_z).wait()

    plsc.subcore_barrier()
    phase(bufs[0], bufs[1], 2, first=True)
    phase(bufs[1], bufs[0], 3)

    def pair(i, carry):
        t0 = 2 * i
        phase(bufs[0], bufs[1], t0 + 2)
        phase(bufs[1], bufs[0], t0 + 3)
        return carry

    lax.fori_loop(1, PAIRS, pair, 0)

    gather_wait(bufs[0])
    col_snapshot(bufs[0])
    compute(bufs[0])
    scatter_wait(bufs[1])
    scatter_start(bufs[0])
    idx_wait(bufs[1])
    scatter_wait(bufs[0])

    plsc.subcore_barrier()
    pltpu.sync_copy(acc_sh.at[pl.ds(s * RPT, RPT)],
                    out_hbm.at[c, pl.ds(s * RPT, RPT)])

    @pl.when(s == 0)
    def _write_tail():
        pltpu.sync_copy(acc_sh.at[pl.ds(NS * RPT, TAIL)],
                        out_hbm.at[c, pl.ds(NS * RPT, TAIL)])


@functools.partial(
    pl.kernel,
    out_type=jax.ShapeDtypeStruct((NC, N, D), jnp.float32),
    mesh=plsc.VectorSubcoreMesh(core_axis_name="c", subcore_axis_name="s"),
    scratch_types=[
        pltpu.VMEM((K,), jnp.int32),
        pltpu.VMEM((K,), jnp.int32),
        pltpu.VMEM((K, D), jnp.float32),
        pltpu.VMEM((K, D), jnp.float32),
        pltpu.VMEM((K,), jnp.int32),
        pltpu.VMEM((K,), jnp.int32),
        pltpu.VMEM((K, D), jnp.float32),
        pltpu.VMEM((K, D), jnp.float32),
        pltpu.VMEM((K,), jnp.int32),
        pltpu.VMEM((K,), jnp.int32),
        pltpu.VMEM((ZR, D), jnp.float32),
        pltpu.VMEM_SHARED((N, D), jnp.float32),
        pltpu.SemaphoreType.DMA,
        pltpu.SemaphoreType.DMA,
        pltpu.SemaphoreType.DMA,
        pltpu.SemaphoreType.DMA,
        pltpu.SemaphoreType.DMA,
        pltpu.SemaphoreType.DMA,
        pltpu.SemaphoreType.DMA,
    ],
)
def _message_pass(h_hbm, row_hbm, col_hbm, out_hbm,
                  row_a, col_a, src_a, tgt_a, row_b, col_b, src_b, tgt_b,
                  scol_a, scol_b, zero_v, acc_sh,
                  sem_ia, sem_ib, sem_ga, sem_gb, sem_sa, sem_sb, sem_z):
    _mp_body(h_hbm, row_hbm, col_hbm, out_hbm,
             row_a, col_a, src_a, tgt_a, row_b, col_b, src_b, tgt_b,
             scol_a, scol_b, zero_v, acc_sh,
             sem_ia, sem_ib, sem_ga, sem_gb, sem_sa, sem_sb, sem_z)


def kernel(x, edge_index, W_in, b_in, W1, b1, W2, b2, Wc1, bc1, Wc2, bc2):
    ei = edge_index.astype(jnp.int32)
    pad = jnp.zeros((EPAD,), jnp.int32)
    row = jnp.concatenate([ei[0], pad])
    col = jnp.concatenate([ei[1], pad])

    h0 = _dense_in(x, W_in.T, b_in.reshape(1, -1))
    p = _message_pass(h0, row, col)
    h1 = _dense_mid(p[0], p[1], W1.T, b1.reshape(1, -1))
    p = _message_pass(h1, row, col)
    return _head(p[0], p[1], W2.T, b2.reshape(1, -1), Wc1.T,
                 bc1.reshape(1, -1), Wc2.T, bc2.reshape(1, -1))
